# baseline (device time: 51420 ns/iter reference)
import functools

import jax
import jax.numpy as jnp
from jax import lax
from jax.experimental import pallas as pl
from jax.experimental.pallas import tpu as pltpu

NX, NY, NZ = 2, 4, 4
M_TOT, D = 4096, 1024
M_PER = M_TOT // NY
NSUB = NX * NZ
SUB = M_PER // NSUB


def kernel(partial, gamma):
    sub_base = (lax.axis_index("x") * NZ + lax.axis_index("z")) * SUB
    p4 = lax.dynamic_slice(
        partial.reshape(NY, M_PER, D), (0, sub_base, 0), (NY, SUB, D)
    ).reshape(NY * SUB, D)
    gamma = gamma.reshape(1, D)

    def body(pbuf, g_ref, out_ref,
             ycomm, ysend, yrecv,
             zcomm, zsend, zrecv,
             xbuf, xsend, xrecv):
        x = lax.axis_index("x")
        j = lax.axis_index("y")
        z = lax.axis_index("z")
        y_r = (j + 1) % NY
        z_r = (z + 1) % NZ
        x_p = 1 - x
        sub_base = (x * NZ + z) * SUB

        barrier_sem = pltpu.get_barrier_semaphore()
        peers = (
            (x, (j - 1) % NY, z), (x, y_r, z),
            (x, j, (z - 1) % NZ), (x, j, z_r),
            (x_p, j, z),
        )
        for dev in peers:
            pl.semaphore_signal(
                barrier_sem, inc=1,
                device_id=dev, device_id_type=pl.DeviceIdType.MESH,
            )
        pl.semaphore_wait(barrier_sem, len(peers))

        ycomm[0] = pbuf[pl.ds(((j - 1) % NY) * SUB, SUB), :].astype(jnp.bfloat16)
        acc = None
        for h in range(NY - 1):
            rdma = pltpu.make_async_remote_copy(
                src_ref=ycomm.at[h],
                dst_ref=ycomm.at[h + 1],
                send_sem=ysend.at[h],
                recv_sem=yrecv.at[h],
                device_id=(x, y_r, z),
                device_id_type=pl.DeviceIdType.MESH,
            )
            rdma.start()
            rdma.wait()
            c = (j - 2 - h) % NY
            acc = (
                ycomm[h + 1].astype(jnp.float32)
                + pbuf[pl.ds(c * SUB, SUB), :]
            )
            if h < NY - 2:
                ycomm[h + 1] = acc.astype(jnp.bfloat16)

        rms = jnp.sqrt(jnp.mean(acc * acc, axis=-1, keepdims=True) + 1e-6)
        res = acc / rms * g_ref[...]
        out_ref[pl.ds(sub_base, SUB), :] = res
        zcomm[0] = res.astype(jnp.bfloat16)

        for h in range(NZ - 1):
            rdma = pltpu.make_async_remote_copy(
                src_ref=zcomm.at[h],
                dst_ref=zcomm.at[h + 1],
                send_sem=zsend.at[h],
                recv_sem=zrecv.at[h],
                device_id=(x, j, z_r),
                device_id_type=pl.DeviceIdType.MESH,
            )
            rdma.start()
            rdma.wait()

        xr = pltpu.make_async_remote_copy(
            src_ref=zcomm,
            dst_ref=xbuf,
            send_sem=xsend.at[0],
            recv_sem=xrecv.at[0],
            device_id=(x_p, j, z),
            device_id_type=pl.DeviceIdType.MESH,
        )
        xr.start()
        xr.wait()

        for t in range(NZ):
            zo = (z - t) % NZ
            if t > 0:
                out_ref[pl.ds((x * NZ + zo) * SUB, SUB), :] = (
                    zcomm[t].astype(jnp.float32)
                )
            out_ref[pl.ds((x_p * NZ + zo) * SUB, SUB), :] = (
                xbuf[t].astype(jnp.float32)
            )

        @functools.partial(
            pl.run_scoped, second_barrier=pltpu.SemaphoreType.REGULAR
        )
        def _(second_barrier):
            for dev in peers:
                pl.semaphore_signal(
                    second_barrier, inc=1,
                    device_id=dev, device_id_type=pl.DeviceIdType.MESH,
                )
            pl.semaphore_wait(second_barrier, len(peers))

    return pl.pallas_call(
        body,
        out_shape=jax.ShapeDtypeStruct((M_PER, D), jnp.float32),
        in_specs=[
            pl.BlockSpec(memory_space=pltpu.VMEM),
            pl.BlockSpec(memory_space=pltpu.VMEM),
        ],
        out_specs=pl.BlockSpec(memory_space=pltpu.VMEM),
        scratch_shapes=[
            pltpu.VMEM((NY, SUB, D), jnp.bfloat16),
            pltpu.SemaphoreType.DMA((NY - 1,)),
            pltpu.SemaphoreType.DMA((NY - 1,)),
            pltpu.VMEM((NZ, SUB, D), jnp.bfloat16),
            pltpu.SemaphoreType.DMA((NZ - 1,)),
            pltpu.SemaphoreType.DMA((NZ - 1,)),
            pltpu.VMEM((NZ, SUB, D), jnp.bfloat16),
            pltpu.SemaphoreType.DMA((NZ,)),
            pltpu.SemaphoreType.DMA((NZ,)),
        ],
        compiler_params=pltpu.CompilerParams(collective_id=0),
    )(p4, gamma)


# device time: 42752 ns/iter; 1.2028x vs baseline; 1.2028x over previous
import functools

import jax
import jax.numpy as jnp
from jax import lax
from jax.experimental import pallas as pl
from jax.experimental.pallas import tpu as pltpu

NX, NY, NZ = 2, 4, 4
M_TOT, D = 4096, 1024
M_PER = M_TOT // NY
NSUB = NX * NZ
SUB = M_PER // NSUB


def kernel(partial, gamma):
    sub_base = (lax.axis_index("x") * NZ + lax.axis_index("z")) * SUB
    p4 = lax.dynamic_slice(
        partial.reshape(NY, M_PER, D), (0, sub_base, 0), (NY, SUB, D)
    ).reshape(NY * SUB, D)
    gamma = gamma.reshape(1, D)

    def body(pbuf, g_ref, out_ref,
             ycomm, ysend, yrecv,
             zcomm, zsend, zrecv,
             xbuf, xsend, xrecv):
        x = lax.axis_index("x")
        j = lax.axis_index("y")
        z = lax.axis_index("z")
        y_r = (j + 1) % NY
        z_r = (z + 1) % NZ
        x_p = 1 - x
        sub_base = (x * NZ + z) * SUB

        barrier_sem = pltpu.get_barrier_semaphore()
        peers = (
            (x, (j - 1) % NY, z), (x, y_r, z),
            (x, j, (z - 1) % NZ), (x, j, z_r),
            (x_p, j, z),
        )
        for dev in peers:
            pl.semaphore_signal(
                barrier_sem, inc=1,
                device_id=dev, device_id_type=pl.DeviceIdType.MESH,
            )
        pl.semaphore_wait(barrier_sem, len(peers))

        ycomm[0] = pbuf[pl.ds(((j - 1) % NY) * SUB, SUB), :].astype(jnp.bfloat16)
        acc = None
        for h in range(NY - 1):
            rdma = pltpu.make_async_remote_copy(
                src_ref=ycomm.at[h],
                dst_ref=ycomm.at[h + 1],
                send_sem=ysend.at[h],
                recv_sem=yrecv.at[h],
                device_id=(x, y_r, z),
                device_id_type=pl.DeviceIdType.MESH,
            )
            rdma.start()
            rdma.wait()
            c = (j - 2 - h) % NY
            acc = (
                ycomm[h + 1].astype(jnp.float32)
                + pbuf[pl.ds(c * SUB, SUB), :]
            )
            if h < NY - 2:
                ycomm[h + 1] = acc.astype(jnp.bfloat16)

        rms = jnp.sqrt(jnp.mean(acc * acc, axis=-1, keepdims=True) + 1e-6)
        res = acc / rms * g_ref[...]
        out_ref[pl.ds(sub_base, SUB), :] = res
        zcomm[0] = res.astype(jnp.bfloat16)

        def x_forward(t):
            xr = pltpu.make_async_remote_copy(
                src_ref=zcomm.at[t],
                dst_ref=xbuf.at[t],
                send_sem=xsend.at[t],
                recv_sem=xrecv.at[t],
                device_id=(x_p, j, z),
                device_id_type=pl.DeviceIdType.MESH,
            )
            xr.start()
            return xr

        xfwd = [x_forward(0)]
        for h in range(NZ - 1):
            rdma = pltpu.make_async_remote_copy(
                src_ref=zcomm.at[h],
                dst_ref=zcomm.at[h + 1],
                send_sem=zsend.at[h],
                recv_sem=zrecv.at[h],
                device_id=(x, j, z_r),
                device_id_type=pl.DeviceIdType.MESH,
            )
            rdma.start()
            xfwd[h].wait_send()
            rdma.wait()
            xfwd.append(x_forward(h + 1))
        xfwd[NZ - 1].wait_send()

        for t in range(NZ):
            zo = (z - t) % NZ
            if t > 0:
                out_ref[pl.ds((x * NZ + zo) * SUB, SUB), :] = (
                    zcomm[t].astype(jnp.float32)
                )
            xfwd[t].wait_recv()
            out_ref[pl.ds((x_p * NZ + zo) * SUB, SUB), :] = (
                xbuf[t].astype(jnp.float32)
            )

        @functools.partial(
            pl.run_scoped, second_barrier=pltpu.SemaphoreType.REGULAR
        )
        def _(second_barrier):
            for dev in peers:
                pl.semaphore_signal(
                    second_barrier, inc=1,
                    device_id=dev, device_id_type=pl.DeviceIdType.MESH,
                )
            pl.semaphore_wait(second_barrier, len(peers))

    return pl.pallas_call(
        body,
        out_shape=jax.ShapeDtypeStruct((M_PER, D), jnp.float32),
        in_specs=[
            pl.BlockSpec(memory_space=pltpu.VMEM),
            pl.BlockSpec(memory_space=pltpu.VMEM),
        ],
        out_specs=pl.BlockSpec(memory_space=pltpu.VMEM),
        scratch_shapes=[
            pltpu.VMEM((NY, SUB, D), jnp.bfloat16),
            pltpu.SemaphoreType.DMA((NY - 1,)),
            pltpu.SemaphoreType.DMA((NY - 1,)),
            pltpu.VMEM((NZ, SUB, D), jnp.bfloat16),
            pltpu.SemaphoreType.DMA((NZ - 1,)),
            pltpu.SemaphoreType.DMA((NZ - 1,)),
            pltpu.VMEM((NZ, SUB, D), jnp.bfloat16),
            pltpu.SemaphoreType.DMA((NZ,)),
            pltpu.SemaphoreType.DMA((NZ,)),
        ],
        compiler_params=pltpu.CompilerParams(collective_id=0),
    )(p4, gamma)
